# baseline (speedup 1.0000x reference)
import functools

import jax
import jax.numpy as jnp
from jax import lax
from jax.experimental import pallas as pl
from jax.experimental.pallas import tpu as pltpu
from jax.experimental.pallas import tpu_sc as plsc

_VOCAB = 38462 * 26
_EMBED_DIM = 16
_BATCH = 16384
_N_FIELDS = 26
_N_IDX = _BATCH * _N_FIELDS

_INFO = plsc.get_sparse_core_info()
_NC = _INFO.num_cores
_NS = _INFO.num_subcores
_NW = _NC * _NS
_CHUNK = 128
_PER_W = _N_IDX // _NW
_CPW = _PER_W // _CHUNK


@functools.partial(
    pl.kernel,
    mesh=plsc.VectorSubcoreMesh(core_axis_name="c", subcore_axis_name="s"),
    out_type=jax.ShapeDtypeStruct((_N_IDX, _EMBED_DIM), jnp.float32),
    scratch_types=[
        pltpu.VMEM((_CPW, _CHUNK), jnp.int32),
        pltpu.VMEM((_CHUNK, _EMBED_DIM), jnp.float32),
        pltpu.SemaphoreType.DMA,
    ],
    compiler_params=pltpu.CompilerParams(use_tc_tiling_on_sc=False),
)
def _sc_gather(idx_hbm, table_hbm, out_hbm, idx_v, rows_v, sem):
    wid = lax.axis_index("s") * _NC + lax.axis_index("c")
    pltpu.sync_copy(idx_hbm.at[wid], idx_v)

    def chunk(j, carry):
        pltpu.async_copy(table_hbm.at[idx_v.at[j]], rows_v, sem).wait()
        base = (wid * _CPW + j) * _CHUNK
        pltpu.sync_copy(rows_v, out_hbm.at[pl.ds(base, _CHUNK)])
        return carry

    lax.fori_loop(0, _CPW, chunk, 0)


def kernel(x, table):
    idx = x.astype(jnp.int32).reshape(_NW, _CPW, _CHUNK)
    out = _sc_gather(idx, table)
    return out.reshape(_BATCH, _N_FIELDS, _EMBED_DIM)


# speedup vs baseline: 1.0729x; 1.0729x over previous
import functools

import jax
import jax.numpy as jnp
from jax import lax
from jax.experimental import pallas as pl
from jax.experimental.pallas import tpu as pltpu
from jax.experimental.pallas import tpu_sc as plsc

_VOCAB = 38462 * 26
_EMBED_DIM = 16
_BATCH = 16384
_N_FIELDS = 26
_N_IDX = _BATCH * _N_FIELDS

_INFO = plsc.get_sparse_core_info()
_NC = _INFO.num_cores
_NS = _INFO.num_subcores
_NW = _NC * _NS
_CHUNK = 128
_PER_W = _N_IDX // _NW
_CPW = _PER_W // _CHUNK
_NCH = 13
_NSUP = _CPW // _NCH
_ROWS = _NCH * _CHUNK


@functools.partial(
    pl.kernel,
    mesh=plsc.VectorSubcoreMesh(core_axis_name="c", subcore_axis_name="s"),
    out_type=jax.ShapeDtypeStruct((_N_IDX, _EMBED_DIM), jnp.float32),
    scratch_types=[
        pltpu.VMEM((_CPW, _CHUNK), jnp.int32),
        pltpu.VMEM((2, _ROWS, _EMBED_DIM), jnp.float32),
        pltpu.SemaphoreType.DMA,
        pltpu.SemaphoreType.DMA((2,)),
    ],
    compiler_params=pltpu.CompilerParams(use_tc_tiling_on_sc=False),
)
def _sc_gather(idx_hbm, table_hbm, out_hbm, idx_v, rows_v, gsem, wsem):
    wid = lax.axis_index("s") * _NC + lax.axis_index("c")
    pltpu.sync_copy(idx_hbm.at[wid], idx_v)

    def super_it(s, carry):
        p = lax.rem(s, 2)

        @pl.when(s >= 2)
        def _():
            pltpu.make_async_copy(
                rows_v.at[p], out_hbm.at[pl.ds(0, _ROWS)], wsem.at[p]
            ).wait()

        descs = []
        for b in range(_NCH):
            descs.append(pltpu.async_copy(
                table_hbm.at[idx_v.at[s * _NCH + b]],
                rows_v.at[p, pl.ds(b * _CHUNK, _CHUNK)],
                gsem,
            ))
        for d in descs:
            d.wait()

        base = (wid * _CPW + s * _NCH) * _CHUNK
        pltpu.async_copy(rows_v.at[p], out_hbm.at[pl.ds(base, _ROWS)], wsem.at[p])
        return carry

    lax.fori_loop(0, _NSUP, super_it, 0)

    pltpu.make_async_copy(rows_v.at[0], out_hbm.at[pl.ds(0, _ROWS)], wsem.at[0]).wait()
    pltpu.make_async_copy(rows_v.at[1], out_hbm.at[pl.ds(0, _ROWS)], wsem.at[1]).wait()


def kernel(x, table):
    idx = x.astype(jnp.int32).reshape(_NW, _CPW, _CHUNK)
    out = _sc_gather(idx, table)
    return out.reshape(_BATCH, _N_FIELDS, _EMBED_DIM)


# speedup vs baseline: 1.6020x; 1.4932x over previous
import functools

import jax
import jax.numpy as jnp
from jax import lax
from jax.experimental import pallas as pl
from jax.experimental.pallas import tpu as pltpu
from jax.experimental.pallas import tpu_sc as plsc

_VOCAB = 38462 * 26
_D = 16
_B = 16384
_F = 26

_INFO = plsc.get_sparse_core_info()
_NC = _INFO.num_cores
_NS = _INFO.num_subcores
_NW = _NC * _NS
_BC = 1024
_NT = _F * (_B // _BC)
_TPW = _NT // _NW
_SUB = 128
_NSUB = _BC // _SUB


@functools.partial(
    pl.kernel,
    mesh=plsc.VectorSubcoreMesh(core_axis_name="c", subcore_axis_name="s"),
    out_type=jax.ShapeDtypeStruct((_F, _D, _B), jnp.float32),
    scratch_types=[
        pltpu.VMEM((_BC,), jnp.int32),
        pltpu.VMEM((2, _SUB, _D), jnp.float32),
        pltpu.VMEM((_D, _BC), jnp.float32),
        pltpu.SemaphoreType.DMA,
        pltpu.SemaphoreType.DMA,
        pltpu.SemaphoreType.DMA,
    ],
    compiler_params=pltpu.CompilerParams(
        use_tc_tiling_on_sc=False, needs_layout_passes=False),
)
def _sc_embed(xT_hbm, tab_hbm, outT_hbm, idx_v, rows_v, tbuf, gsem0, gsem1, wsem):
    wid = lax.axis_index("s") * _NC + lax.axis_index("c")
    iota16 = lax.iota(jnp.int32, 16)
    gsems = (gsem0, gsem1)

    def task(t_local, carry):
        t = wid * _TPW + t_local
        f = t // (_B // _BC)
        c = lax.rem(t, _B // _BC)

        pltpu.sync_copy(xT_hbm.at[f, pl.ds(c * _BC, _BC)], idx_v)

        descs = [None] * _NSUB
        descs[0] = pltpu.async_copy(
            tab_hbm.at[idx_v.at[pl.ds(0, _SUB)]], rows_v.at[0], gsems[0])

        @pl.when(t_local > 0)
        def _():
            pltpu.make_async_copy(
                tbuf, outT_hbm.at[0, :, pl.ds(0, _BC)], wsem).wait()

        for s in range(_NSUB):
            p = s % 2
            if s + 1 < _NSUB:
                descs[s + 1] = pltpu.async_copy(
                    tab_hbm.at[idx_v.at[pl.ds((s + 1) * _SUB, _SUB)]],
                    rows_v.at[1 - p], gsems[(s + 1) % 2])
            descs[s].wait()
            for g in range(_SUB // 16):
                ridx = iota16 + (g * 16)
                for e in range(_D):
                    vals = plsc.load_gather(
                        rows_v,
                        [jnp.full((16,), p, jnp.int32), ridx,
                         jnp.full((16,), e, jnp.int32)])
                    tbuf[e, pl.ds(s * _SUB + g * 16, 16)] = vals

        pltpu.async_copy(tbuf, outT_hbm.at[f, :, pl.ds(c * _BC, _BC)], wsem)
        return carry

    lax.fori_loop(0, _TPW, task, 0)
    pltpu.make_async_copy(tbuf, outT_hbm.at[0, :, pl.ds(0, _BC)], wsem).wait()


def kernel(x, table):
    xT = x.T.astype(jnp.int32)
    outT = _sc_embed(xT, table)
    return outT.transpose(2, 0, 1)
